# baseline (speedup 1.0000x reference)
import functools

import jax
import jax.numpy as jnp
from jax import lax
from jax.experimental import pallas as pl
from jax.experimental.pallas import tpu as pltpu
from jax.experimental.pallas import tpu_sc as plsc

_L = 16
_CH = 128


def _make_gather(B):
    info = plsc.get_sparse_core_info()
    NC, NS = info.num_cores, info.num_subcores
    NW = NC * NS
    assert B % NW == 0
    b_per_w = B // NW
    rw = b_per_w * 4
    tw = b_per_w * 3
    assert rw % _CH == 0 and tw % _CH == 0

    mesh = plsc.VectorSubcoreMesh(core_axis_name="c", subcore_axis_name="s")

    @functools.partial(
        pl.kernel,
        mesh=mesh,
        compiler_params=pltpu.CompilerParams(
            use_tc_tiling_on_sc=False, needs_layout_passes=False),
        out_type=[
            jax.ShapeDtypeStruct((B * 4,), jnp.float32),
            jax.ShapeDtypeStruct((B * 3,), jnp.float32),
        ],
        scratch_types=[
            pltpu.VMEM((b_per_w,), jnp.int32),
            pltpu.VMEM((rw,), jnp.int32),
            pltpu.VMEM((tw,), jnp.int32),
            pltpu.VMEM((rw,), jnp.float32),
            pltpu.VMEM((tw,), jnp.float32),
            pltpu.SemaphoreType.DMA,
        ],
    )
    def gather_kernel(idx_hbm, rot_hbm, trans_hbm, rot_out, trans_out,
                      idx_v, roff_v, toff_v, rdat_v, tdat_v, sem):
        wid = lax.axis_index("s") * NC + lax.axis_index("c")
        base = wid * b_per_w
        pltpu.sync_copy(idx_hbm.at[pl.ds(base, b_per_w)], idx_v)

        lane = lax.iota(jnp.int32, _L)

        def rot_body(i, carry):
            w0 = i * _L
            jj = lane + w0
            q = jj >> 2
            r = jj & 3
            g = plsc.load_gather(idx_v, [q])
            roff_v[pl.ds(w0, _L)] = g * 4 + r
            return carry

        lax.fori_loop(0, rw // _L, rot_body, 0)

        def tr_body(i, carry):
            w0 = i * _L
            jj = lane + w0
            q = jj // 3
            r = jj - q * 3
            g = plsc.load_gather(idx_v, [q])
            toff_v[pl.ds(w0, _L)] = g * 3 + r
            return carry

        lax.fori_loop(0, tw // _L, tr_body, 0)

        copies = []
        for c in range(rw // _CH):
            copies.append(pltpu.async_copy(
                rot_hbm.at[roff_v.at[pl.ds(c * _CH, _CH)]],
                rdat_v.at[pl.ds(c * _CH, _CH)], sem))
        for c in range(tw // _CH):
            copies.append(pltpu.async_copy(
                trans_hbm.at[toff_v.at[pl.ds(c * _CH, _CH)]],
                tdat_v.at[pl.ds(c * _CH, _CH)], sem))
        for cp in copies:
            cp.wait()

        pltpu.sync_copy(rdat_v, rot_out.at[pl.ds(base * 4, rw)])
        pltpu.sync_copy(tdat_v, trans_out.at[pl.ds(base * 3, tw)])

    return gather_kernel


def kernel(camera_idx, rotations, translations):
    B = camera_idx.shape[0]
    gather_kernel = _make_gather(B)
    rot_o, trans_o = gather_kernel(
        camera_idx.astype(jnp.int32),
        rotations.reshape(-1),
        translations.reshape(-1),
    )
    return rot_o.reshape(B, 4), trans_o.reshape(B, 3)


# speedup vs baseline: 6.4591x; 6.4591x over previous
import functools

import jax
import jax.numpy as jnp
from jax import lax
from jax.experimental import pallas as pl
from jax.experimental.pallas import tpu as pltpu
from jax.experimental.pallas import tpu_sc as plsc

_L = 16
_CH = 128


def _make_gather(B, V):
    info = plsc.get_sparse_core_info()
    NC, NS = info.num_cores, info.num_subcores
    NW = NC * NS
    assert B % (NW * _CH) == 0
    bpw = B // NW
    ng = bpw // _L
    nb = bpw // _CH

    mesh = plsc.VectorSubcoreMesh(core_axis_name="c", subcore_axis_name="s")

    @functools.partial(
        pl.kernel,
        mesh=mesh,
        compiler_params=pltpu.CompilerParams(
            use_tc_tiling_on_sc=False,
            needs_layout_passes=False,
            disable_bounds_checks=True,
        ),
        out_type=[
            jax.ShapeDtypeStruct((B * 4,), jnp.float32),
            jax.ShapeDtypeStruct((B * 4,), jnp.float32),
        ],
        scratch_types=[
            pltpu.VMEM((bpw,), jnp.int32),
            pltpu.VMEM((4 * bpw,), jnp.int32),
            pltpu.VMEM((3 * bpw,), jnp.int32),
            pltpu.VMEM((4 * bpw,), jnp.float32),
            pltpu.VMEM((4 * bpw,), jnp.float32),
            pltpu.SemaphoreType.DMA,
        ],
    )
    def gather_kernel(idx_hbm, rot_hbm, trans_hbm, rot_out, trans_out,
                      idx_v, roff_v, toff_v, rimg_v, timg_v, sem):
        wid = lax.axis_index("s") * NC + lax.axis_index("c")
        base = wid * bpw
        pltpu.sync_copy(idx_hbm.at[pl.ds(base, bpw)], idx_v)

        def build(t, carry):
            b0 = t * _L
            i16 = idx_v[pl.ds(b0, _L)]
            for k in range(4):
                roff_v[pl.ds(k * bpw + b0, _L)] = i16 + k * V
            for k in range(3):
                toff_v[pl.ds(k * bpw + b0, _L)] = i16 + k * V
            return carry

        lax.fori_loop(0, ng, build, 0)

        copies = []
        for cb in range(nb):
            for k in range(4):
                copies.append(pltpu.async_copy(
                    rot_hbm.at[roff_v.at[pl.ds(k * bpw + cb * _CH, _CH)]],
                    rimg_v.at[pl.ds(cb * 512 + k * _CH, _CH)], sem))
            for k in range(3):
                copies.append(pltpu.async_copy(
                    trans_hbm.at[toff_v.at[pl.ds(k * bpw + cb * _CH, _CH)]],
                    timg_v.at[pl.ds(cb * 512 + k * _CH, _CH)], sem))
        for cp in copies:
            cp.wait()

        pltpu.sync_copy(rimg_v, rot_out.at[pl.ds(base * 4, 4 * bpw)])
        pltpu.sync_copy(timg_v, trans_out.at[pl.ds(base * 4, 4 * bpw)])

    return gather_kernel


def kernel(camera_idx, rotations, translations):
    B = camera_idx.shape[0]
    V = rotations.shape[0]
    gather_kernel = _make_gather(B, V)
    rot_pl = jnp.swapaxes(rotations, 0, 1).reshape(-1)
    trans_pl = jnp.swapaxes(translations, 0, 1).reshape(-1)
    rot_f, trans_f = gather_kernel(
        camera_idx.astype(jnp.int32), rot_pl, trans_pl)
    rot = rot_f.reshape(B // _CH, 4, _CH).transpose(0, 2, 1).reshape(B, 4)
    trans = trans_f.reshape(B // _CH, 4, _CH).transpose(0, 2, 1).reshape(B, 4)
    return rot, trans[:, :3]
